# Initial kernel scaffold; baseline (speedup 1.0000x reference)
import jax
import jax.numpy as jnp
from jax.experimental import pallas as pl


def kernel(input, raw_weight):
    raise NotImplementedError("write your pallas kernel here")


# speedup vs baseline: 1.9410x; 1.9410x over previous
import functools

import jax
import jax.numpy as jnp
from jax import lax
from jax.experimental import pallas as pl
from jax.experimental.pallas import tpu as pltpu
from jax.experimental.pallas import tpu_sc as plsc

NUM_EMB = 1_000_000
D = 64
L = 16
NC = 2
NS = 16
NW = NC * NS
B = 16384 * 50
B_PER_W = B // NW
CHUNK = 128
N_CHUNK = B_PER_W // CHUNK
NBUF = 2

_MAGIC = 0x5F3759DF


def _rsqrt16(x):
    i = plsc.bitcast(x, jnp.int32)
    i = jnp.int32(_MAGIC) - lax.shift_right_arithmetic(i, jnp.int32(1))
    y = plsc.bitcast(i, jnp.float32)
    for _ in range(3):
        y = y * (1.5 - 0.5 * x * y * y)
    return y


def _hsum_all(x):
    dnums = lax.GatherDimensionNumbers(
        offset_dims=(), collapsed_slice_dims=(0,), start_index_map=(0,))
    for k in (1, 2, 4, 8):
        perm = lax.iota(jnp.int32, L) ^ k
        x = x + lax.gather(x, perm[:, None], dnums, slice_sizes=(1,),
                           mode=lax.GatherScatterMode.PROMISE_IN_BOUNDS)
    return x


def _sc_kernel(idx_hbm, table_hbm, out_hbm, idx_v, gbuf, sbuf,
               gsems, ssems):
    wid = lax.axis_index("s") * NC + lax.axis_index("c")
    base_row = wid * B_PER_W

    pltpu.sync_copy(idx_hbm.at[wid], idx_v)

    def start_gather(b, c):
        pltpu.async_copy(table_hbm.at[idx_v.at[c]], gbuf.at[b], gsems[b])

    def wait_gather(b, c):
        pltpu.make_async_copy(table_hbm.at[idx_v.at[c]], gbuf.at[b],
                              gsems[b]).wait()

    def start_store(b, c):
        pltpu.async_copy(sbuf.at[b], out_hbm.at[pl.ds(base_row + c * CHUNK,
                                                      CHUNK)], ssems[b])

    def wait_store(b, c):
        pltpu.make_async_copy(sbuf.at[b],
                              out_hbm.at[pl.ds(base_row + c * CHUNK, CHUNK)],
                              ssems[b]).wait()

    for b in range(NBUF):
        start_gather(b, b)

    def body(i, carry):
        for b in range(NBUF):
            c = i * NBUF + b
            wait_gather(b, c)

            @pl.when(i > 0)
            def _():
                wait_store(b, c - NBUF)

            def row_body(r, carry2):
                v0 = gbuf[b, r, pl.ds(0, L)]
                v1 = gbuf[b, r, pl.ds(L, L)]
                v2 = gbuf[b, r, pl.ds(2 * L, L)]
                v3 = gbuf[b, r, pl.ds(3 * L, L)]
                ss = v0 * v0 + v1 * v1 + v2 * v2 + v3 * v3
                m = _hsum_all(ss) * (1.0 / D) + 1e-30
                y = _rsqrt16(m)
                sbuf[b, r, pl.ds(0, L)] = v0 * y
                sbuf[b, r, pl.ds(L, L)] = v1 * y
                sbuf[b, r, pl.ds(2 * L, L)] = v2 * y
                sbuf[b, r, pl.ds(3 * L, L)] = v3 * y
                return carry2

            lax.fori_loop(0, CHUNK, row_body, 0)
            start_store(b, c)

            @pl.when(c + NBUF < N_CHUNK)
            def _():
                start_gather(b, c + NBUF)
        return carry

    lax.fori_loop(0, N_CHUNK // NBUF, body, 0)
    for b in range(NBUF):
        wait_store(b, N_CHUNK - NBUF + b)


@jax.jit
def _run(idx, table):
    mesh = plsc.VectorSubcoreMesh(core_axis_name="c", subcore_axis_name="s")
    f = functools.partial(
        pl.kernel,
        mesh=mesh,
        compiler_params=pltpu.CompilerParams(needs_layout_passes=False,
                                             use_tc_tiling_on_sc=False),
        out_type=jax.ShapeDtypeStruct((B, D), jnp.float32),
        scratch_types=[
            pltpu.VMEM((N_CHUNK, CHUNK), jnp.int32),
            pltpu.VMEM((NBUF, CHUNK, D), jnp.float32),
            pltpu.VMEM((NBUF, CHUNK, D), jnp.float32),
            [pltpu.SemaphoreType.DMA] * NBUF,
            [pltpu.SemaphoreType.DMA] * NBUF,
        ],
    )(_sc_kernel)
    return f(idx, table)


def kernel(input, raw_weight):
    idx = input.reshape(NW, N_CHUNK, CHUNK).astype(jnp.int32)
    out = _run(idx, raw_weight)
    return out.reshape(input.shape + (D,))
